# baseline (speedup 1.0000x reference)
import functools

import jax
import jax.numpy as jnp
from jax import lax
from jax.experimental import pallas as pl
from jax.experimental.pallas import tpu as pltpu
from jax.experimental.pallas import tpu_sc as plsc

N = 10000
E = 160000
D = 256
H = 128
NC = 2
NS = 16
CH = 128
IB = 8
NIB = 10
J = NIB * IB
E_PAD = NS * J * CH
NPAD = 10112
ROWS = NPAD // NS
G = 8
DB = 40
RB = 2000
NB = 4


def _deg_body(ei_hbm, zeros_hbm, ones_hbm, out_hbm, idx_v, ones_v, hist_sh,
              ssem):
    c = lax.axis_index("c")
    s = lax.axis_index("s")
    pltpu.sync_copy(zeros_hbm.at[pl.ds(s * ROWS, ROWS)],
                    hist_sh.at[pl.ds(s * ROWS, ROWS)])
    pltpu.sync_copy(ones_hbm, ones_v)
    pltpu.sync_copy(ei_hbm.at[c, s], idx_v)
    plsc.subcore_barrier()

    def block(bb, carry):
        def fire(i, carry2):
            pltpu.async_copy(ones_v, hist_sh.at[idx_v.at[bb * DB + i]], ssem,
                             add=True)
            return carry2

        def drain(i, carry2):
            pltpu.make_async_copy(ones_v, hist_sh.at[idx_v.at[bb * DB + i]],
                                  ssem).wait()
            return carry2

        lax.fori_loop(0, DB, fire, carry)
        return lax.fori_loop(0, DB, drain, carry)

    lax.fori_loop(0, J // DB, block, 0)
    plsc.subcore_barrier()
    pltpu.sync_copy(hist_sh.at[pl.ds(s * ROWS, ROWS)],
                    out_hbm.at[c, pl.ds(s * ROWS, ROWS)])


_deg_call = functools.partial(
    pl.kernel,
    out_type=jax.ShapeDtypeStruct((NC, NPAD, G), jnp.float32),
    mesh=plsc.VectorSubcoreMesh(core_axis_name="c", subcore_axis_name="s"),
    compiler_params=pltpu.CompilerParams(use_tc_tiling_on_sc=False),
    scratch_types=[
        pltpu.VMEM((J, CH), jnp.int32),
        pltpu.VMEM((CH, G), jnp.float32),
        pltpu.VMEM_SHARED((NPAD, G), jnp.float32),
        pltpu.SemaphoreType.DMA,
    ],
)(_deg_body)


def _agg_body(h2_hbm, src_hbm, dst_hbm, zerob_hbm, out_hbm,
              srcb, dstb, gbuf, hq_sh, agg_sh, *sems):
    gsem = sems[0:NB]
    ssem = sems[NB:2 * NB]
    isem = sems[2 * NB:2 * NB + 2]
    c = lax.axis_index("c")
    s = lax.axis_index("s")

    pltpu.sync_copy(h2_hbm.at[pl.ds(c * NPAD + s * ROWS, ROWS)],
                    hq_sh.at[pl.ds(s * ROWS, ROWS)])
    pltpu.sync_copy(zerob_hbm.at[pl.ds(s * ROWS, ROWS)],
                    agg_sh.at[pl.ds(s * ROWS, ROWS)])
    pltpu.async_copy(src_hbm.at[s, 0], srcb.at[0], isem[0])
    pltpu.async_copy(dst_hbm.at[s, 0], dstb.at[0], isem[0])
    plsc.subcore_barrier()

    def block_pair(bb, carry):
        for rb in range(2):
            ib = bb * 2 + rb
            pltpu.make_async_copy(src_hbm.at[s, ib], srcb.at[rb],
                                  isem[rb]).wait()
            pltpu.make_async_copy(dst_hbm.at[s, ib], dstb.at[rb],
                                  isem[rb]).wait()
            for i in range(IB):
                b = i % NB
                pb = (i - 2) % NB
                def wait_scatter(b=b, rb=rb):
                    pltpu.make_async_copy(gbuf.at[b],
                                          agg_sh.at[dstb.at[rb, 0]],
                                          ssem[b]).wait()

                if i >= NB:
                    wait_scatter()
                else:
                    pl.when(ib >= 1)(wait_scatter)
                pltpu.async_copy(hq_sh.at[srcb.at[rb, i]], gbuf.at[b],
                                 gsem[b])

                def fire_prev(pb=pb, rb=rb, i=i):
                    pltpu.make_async_copy(hq_sh.at[srcb.at[rb, 0]],
                                          gbuf.at[pb], gsem[pb]).wait()
                    pidx = (dstb.at[rb, i - 2] if i >= 2
                            else dstb.at[1 - rb, IB - 2 + i])
                    pltpu.async_copy(gbuf.at[pb], agg_sh.at[pidx],
                                     ssem[pb], add=True)

                if i >= 2:
                    fire_prev()
                else:
                    pl.when(ib >= 1)(fire_prev)

                if i == NB - 1:
                    @pl.when(ib < NIB - 1)
                    def _(rb=rb, ib=ib):
                        pltpu.async_copy(src_hbm.at[s, ib + 1],
                                         srcb.at[1 - rb], isem[1 - rb])
                        pltpu.async_copy(dst_hbm.at[s, ib + 1],
                                         dstb.at[1 - rb], isem[1 - rb])
        return carry

    lax.fori_loop(0, NIB // 2, block_pair, 0)
    ls = (NIB - 1) % 2
    for k in (2, 1):
        lb = (IB - k) % NB
        pltpu.make_async_copy(hq_sh.at[srcb.at[ls, IB - k]], gbuf.at[lb],
                              gsem[lb]).wait()
        pltpu.async_copy(gbuf.at[lb], agg_sh.at[dstb.at[ls, IB - k]],
                         ssem[lb], add=True)
    for b in range(NB):
        pltpu.make_async_copy(gbuf.at[b], agg_sh.at[dstb.at[ls, 0]],
                              ssem[b]).wait()
    plsc.subcore_barrier()
    pltpu.sync_copy(agg_sh.at[pl.ds(s * ROWS, ROWS)],
                    out_hbm.at[c, pl.ds(s * ROWS, ROWS)])


_agg_call = functools.partial(
    pl.kernel,
    out_type=jax.ShapeDtypeStruct((NC, NPAD, H), jnp.bfloat16),
    mesh=plsc.VectorSubcoreMesh(core_axis_name="c", subcore_axis_name="s"),
    compiler_params=pltpu.CompilerParams(use_tc_tiling_on_sc=False),
    scratch_types=[
        pltpu.VMEM((2, IB, CH), jnp.int32),
        pltpu.VMEM((2, IB, CH), jnp.int32),
        pltpu.VMEM((NB, CH, H), jnp.bfloat16),
        pltpu.VMEM_SHARED((NPAD, H), jnp.bfloat16),
        pltpu.VMEM_SHARED((NPAD, H), jnp.bfloat16),
    ] + [pltpu.SemaphoreType.DMA] * (2 * NB + 2),
)(_agg_body)


def _prescale_body(feats_ref, deg_ref, out_ref):
    norm = lax.rsqrt(jnp.maximum(deg_ref[...], 1.0))
    out_ref[...] = (feats_ref[...] * norm)[None].astype(jnp.bfloat16)


def _prescale_call(feats, deg_out):
    return pl.pallas_call(
        _prescale_body,
        out_shape=jax.ShapeDtypeStruct((NC, NPAD, H), jnp.bfloat16),
        grid=(NC, N // RB),
        in_specs=[
            pl.BlockSpec((RB, H), lambda c, i: (i, c)),
            pl.BlockSpec((RB, 1), lambda c, i: (i, 0)),
        ],
        out_specs=pl.BlockSpec((1, RB, H), lambda c, i: (c, i, 0)),
    )(feats, deg_out)


def _final_body(agg_ref, deg_ref, w_ref, b_ref, out_ref):
    norm = lax.rsqrt(jnp.maximum(deg_ref[...], 1.0))
    a0 = agg_ref[0]
    a1 = agg_ref[1]
    acc = jnp.dot(a0, w_ref[0:H, :], preferred_element_type=jnp.float32)
    acc += jnp.dot(a1, w_ref[H:D, :], preferred_element_type=jnp.float32)
    out_ref[...] = acc * norm + b_ref[...]


def _final_call(agg2, deg_in, W, b2):
    return pl.pallas_call(
        _final_body,
        out_shape=jax.ShapeDtypeStruct((N, D), jnp.float32),
        grid=(N // RB,),
        in_specs=[
            pl.BlockSpec((NC, RB, H), lambda i: (0, i, 0)),
            pl.BlockSpec((RB, 1), lambda i: (i, 0)),
            pl.BlockSpec((D, D), lambda i: (0, 0)),
            pl.BlockSpec((1, D), lambda i: (0, 0)),
        ],
        out_specs=pl.BlockSpec((RB, D), lambda i: (i, 0)),
    )(agg2, deg_in, W, b2)


@jax.jit
def kernel(feats, edge_index, W, b):
    src = edge_index[0]
    dst = edge_index[1]
    pad = jnp.full((E_PAD - E,), N, jnp.int32)
    src_p = jnp.concatenate([src, pad])
    dst_p = jnp.concatenate([dst, pad])
    ei_deg = jnp.stack([src_p, dst_p]).reshape(NC, NS, J, CH)
    src_r = src_p.reshape(NS, NIB, IB, CH)
    dst_r = dst_p.reshape(NS, NIB, IB, CH)
    zeros_f = jnp.zeros((NPAD, G), jnp.float32)
    zeros_b = jnp.zeros((NPAD, H), jnp.bfloat16)
    ones_h = jnp.ones((CH, G), jnp.float32)

    degv = _deg_call(ei_deg, zeros_f, ones_h)
    deg_out = degv[0, :, 0:1]
    deg_in = degv[1, :, 0:1]

    h2 = _prescale_call(feats, deg_out)
    agg2 = _agg_call(h2.reshape(NC * NPAD, H), src_r, dst_r, zeros_b)
    return _final_call(agg2.astype(jnp.float32), deg_in, W, b.reshape(1, D))


# speedup vs baseline: 1.0556x; 1.0556x over previous
import functools

import jax
import jax.numpy as jnp
from jax import lax
from jax.experimental import pallas as pl
from jax.experimental.pallas import tpu as pltpu
from jax.experimental.pallas import tpu_sc as plsc

N = 10000
E = 160000
D = 256
H = 128
NC = 2
NS = 16
CH = 128
IB = 8
NIB = 10
J = NIB * IB
E_PAD = NS * J * CH
NPAD = 10112
ROWS = NPAD // NS
G = 8
DB = 40
RB = 2000
NB = 4


def _deg_body(ei_hbm, zeros_hbm, ones_hbm, out_hbm, idx_v, ones_v, hist_sh,
              ssem):
    c = lax.axis_index("c")
    s = lax.axis_index("s")
    pltpu.sync_copy(zeros_hbm.at[pl.ds(s * ROWS, ROWS)],
                    hist_sh.at[pl.ds(s * ROWS, ROWS)])
    pltpu.sync_copy(ones_hbm, ones_v)
    pltpu.sync_copy(ei_hbm.at[c, s], idx_v)
    plsc.subcore_barrier()

    def block(bb, carry):
        def fire(i, carry2):
            pltpu.async_copy(ones_v, hist_sh.at[idx_v.at[bb * DB + i]], ssem,
                             add=True)
            return carry2

        def drain(i, carry2):
            pltpu.make_async_copy(ones_v, hist_sh.at[idx_v.at[bb * DB + i]],
                                  ssem).wait()
            return carry2

        lax.fori_loop(0, DB, fire, carry)
        return lax.fori_loop(0, DB, drain, carry)

    lax.fori_loop(0, J // DB, block, 0)
    plsc.subcore_barrier()
    pltpu.sync_copy(hist_sh.at[pl.ds(s * ROWS, ROWS)],
                    out_hbm.at[c, pl.ds(s * ROWS, ROWS)])


_deg_call = functools.partial(
    pl.kernel,
    out_type=jax.ShapeDtypeStruct((NC, NPAD, G), jnp.float32),
    mesh=plsc.VectorSubcoreMesh(core_axis_name="c", subcore_axis_name="s"),
    compiler_params=pltpu.CompilerParams(use_tc_tiling_on_sc=False),
    scratch_types=[
        pltpu.VMEM((J, CH), jnp.int32),
        pltpu.VMEM((CH, G), jnp.float32),
        pltpu.VMEM_SHARED((NPAD, G), jnp.float32),
        pltpu.SemaphoreType.DMA,
    ],
)(_deg_body)


def _agg_body(h2_hbm, src_hbm, dst_hbm, zerob_hbm, out_hbm,
              srcb, dstb, gbuf, hq_sh, agg_sh, *sems):
    gsem = sems[0:NB]
    ssem = sems[NB:2 * NB]
    isem = sems[2 * NB:2 * NB + 2]
    c = lax.axis_index("c")
    s = lax.axis_index("s")

    pltpu.sync_copy(h2_hbm.at[pl.ds(c * NPAD + s * ROWS, ROWS)],
                    hq_sh.at[pl.ds(s * ROWS, ROWS)])
    pltpu.sync_copy(zerob_hbm.at[pl.ds(s * ROWS, ROWS)],
                    agg_sh.at[pl.ds(s * ROWS, ROWS)])
    pltpu.async_copy(src_hbm.at[s, 0], srcb.at[0], isem[0])
    pltpu.async_copy(dst_hbm.at[s, 0], dstb.at[0], isem[0])
    plsc.subcore_barrier()

    def block_pair(bb, carry):
        for rb in range(2):
            ib = bb * 2 + rb
            pltpu.make_async_copy(src_hbm.at[s, ib], srcb.at[rb],
                                  isem[rb]).wait()
            pltpu.make_async_copy(dst_hbm.at[s, ib], dstb.at[rb],
                                  isem[rb]).wait()
            for i in range(IB):
                b = i % NB
                pb = (i - 2) % NB
                def wait_scatter(b=b, rb=rb):
                    pltpu.make_async_copy(gbuf.at[b],
                                          agg_sh.at[dstb.at[rb, 0]],
                                          ssem[b]).wait()

                if i >= NB:
                    wait_scatter()
                else:
                    pl.when(ib >= 1)(wait_scatter)
                pltpu.async_copy(hq_sh.at[srcb.at[rb, i]], gbuf.at[b],
                                 gsem[b])

                def fire_prev(pb=pb, rb=rb, i=i):
                    pltpu.make_async_copy(hq_sh.at[srcb.at[rb, 0]],
                                          gbuf.at[pb], gsem[pb]).wait()
                    pidx = (dstb.at[rb, i - 2] if i >= 2
                            else dstb.at[1 - rb, IB - 2 + i])
                    pltpu.async_copy(gbuf.at[pb], agg_sh.at[pidx],
                                     ssem[pb], add=True)

                if i >= 2:
                    fire_prev()
                else:
                    pl.when(ib >= 1)(fire_prev)

                if i == NB - 1:
                    @pl.when(ib < NIB - 1)
                    def _(rb=rb, ib=ib):
                        pltpu.async_copy(src_hbm.at[s, ib + 1],
                                         srcb.at[1 - rb], isem[1 - rb])
                        pltpu.async_copy(dst_hbm.at[s, ib + 1],
                                         dstb.at[1 - rb], isem[1 - rb])
        return carry

    lax.fori_loop(0, NIB // 2, block_pair, 0)
    ls = (NIB - 1) % 2
    for k in (2, 1):
        lb = (IB - k) % NB
        pltpu.make_async_copy(hq_sh.at[srcb.at[ls, IB - k]], gbuf.at[lb],
                              gsem[lb]).wait()
        pltpu.async_copy(gbuf.at[lb], agg_sh.at[dstb.at[ls, IB - k]],
                         ssem[lb], add=True)
    for b in range(NB):
        pltpu.make_async_copy(gbuf.at[b], agg_sh.at[dstb.at[ls, 0]],
                              ssem[b]).wait()
    plsc.subcore_barrier()
    pltpu.sync_copy(agg_sh.at[pl.ds(s * ROWS, ROWS)],
                    out_hbm.at[c, pl.ds(s * ROWS, ROWS)])


_agg_call = functools.partial(
    pl.kernel,
    out_type=jax.ShapeDtypeStruct((NC, NPAD, H), jnp.bfloat16),
    mesh=plsc.VectorSubcoreMesh(core_axis_name="c", subcore_axis_name="s"),
    compiler_params=pltpu.CompilerParams(use_tc_tiling_on_sc=False),
    scratch_types=[
        pltpu.VMEM((2, IB, CH), jnp.int32),
        pltpu.VMEM((2, IB, CH), jnp.int32),
        pltpu.VMEM((NB, CH, H), jnp.bfloat16),
        pltpu.VMEM_SHARED((NPAD, H), jnp.bfloat16),
        pltpu.VMEM_SHARED((NPAD, H), jnp.bfloat16),
    ] + [pltpu.SemaphoreType.DMA] * (2 * NB + 2),
)(_agg_body)


def _prescale_body(feats_ref, deg_ref, out_ref):
    norm = lax.rsqrt(jnp.maximum(deg_ref[0, :, 0:1], 1.0))
    out_ref[...] = (feats_ref[...] * norm)[None].astype(jnp.bfloat16)


def _prescale_call(feats, deg_out):
    return pl.pallas_call(
        _prescale_body,
        out_shape=jax.ShapeDtypeStruct((NC, NPAD, H), jnp.bfloat16),
        grid=(NC, N // RB),
        in_specs=[
            pl.BlockSpec((RB, H), lambda c, i: (i, c)),
            pl.BlockSpec((1, RB, G), lambda c, i: (0, i, 0)),
        ],
        out_specs=pl.BlockSpec((1, RB, H), lambda c, i: (c, i, 0)),
    )(feats, deg_out)


def _final_body(agg_ref, deg_ref, w_ref, b_ref, out_ref):
    norm = lax.rsqrt(jnp.maximum(deg_ref[0, :, 0:1], 1.0))
    a0 = agg_ref[0].astype(jnp.float32)
    a1 = agg_ref[1].astype(jnp.float32)
    acc = jnp.dot(a0, w_ref[0:H, :], preferred_element_type=jnp.float32)
    acc += jnp.dot(a1, w_ref[H:D, :], preferred_element_type=jnp.float32)
    out_ref[...] = acc * norm + b_ref[...]


def _final_call(agg2, deg_in, W, b2):
    return pl.pallas_call(
        _final_body,
        out_shape=jax.ShapeDtypeStruct((N, D), jnp.float32),
        grid=(N // RB,),
        in_specs=[
            pl.BlockSpec((NC, RB, H), lambda i: (0, i, 0)),
            pl.BlockSpec((1, RB, G), lambda i: (1, i, 0)),
            pl.BlockSpec((D, D), lambda i: (0, 0)),
            pl.BlockSpec((1, D), lambda i: (0, 0)),
        ],
        out_specs=pl.BlockSpec((RB, D), lambda i: (i, 0)),
    )(agg2, deg_in, W, b2)


@jax.jit
def kernel(feats, edge_index, W, b):
    src = edge_index[0]
    dst = edge_index[1]
    pad = jnp.full((E_PAD - E,), N, jnp.int32)
    src_p = jnp.concatenate([src, pad])
    dst_p = jnp.concatenate([dst, pad])
    ei_deg = jnp.stack([src_p, dst_p]).reshape(NC, NS, J, CH)
    src_r = src_p.reshape(NS, NIB, IB, CH)
    dst_r = dst_p.reshape(NS, NIB, IB, CH)
    zeros_f = jnp.zeros((NPAD, G), jnp.float32)
    zeros_b = jnp.zeros((NPAD, H), jnp.bfloat16)
    ones_h = jnp.ones((CH, G), jnp.float32)

    degv = _deg_call(ei_deg, zeros_f, ones_h)

    h2 = _prescale_call(feats, degv)
    agg2 = _agg_call(h2.reshape(NC * NPAD, H), src_r, dst_r, zeros_b)
    return _final_call(agg2, degv, W, b.reshape(1, D))
